# baseline (speedup 1.0000x reference)
import functools

import jax
import jax.numpy as jnp
from jax import lax
from jax.experimental import pallas as pl
from jax.experimental.pallas import tpu as pltpu, tpu_sc as plsc

NW = 32


def _mesh_wid():
    info = plsc.get_sparse_core_info()
    wid = lax.axis_index("s") * info.num_cores + lax.axis_index("c")
    return wid


def _transpose_call(D, V):
    NP = (V - 1) // 2
    NBLK = (V - 1) // 128
    TAIL = (V - 1) - NBLK * 128
    per_w = NBLK // NW
    extras = NBLK - per_w * NW
    n_pairs_iters = per_w // 2
    mesh = plsc.VectorSubcoreMesh(core_axis_name="c", subcore_axis_name="s")

    @functools.partial(
        pl.kernel,
        mesh=mesh,
        out_type=jax.ShapeDtypeStruct((NP, 2 * D), jnp.float32),
        compiler_params=pltpu.CompilerParams(needs_layout_passes=False),
        scratch_types=[
            pltpu.VMEM((2, D, 128), jnp.float32),
            pltpu.VMEM((2, 64, 128), jnp.float32),
            pltpu.SemaphoreType.DMA,
            pltpu.SemaphoreType.DMA,
            pltpu.SemaphoreType.DMA,
            pltpu.SemaphoreType.DMA,
        ],
    )
    def ka(tabT, tailT, I, Sg, T, r0, r1, w0, w1):
        rsem = (r0, r1)
        wsem = (w0, w1)
        wid = _mesh_wid()
        j0 = wid * per_w

        def fire_read(j, b):
            pltpu.async_copy(tabT.at[:, pl.ds(j * 128, 128)], Sg.at[b], rsem[b])

        def wait_read(b):
            pltpu.make_async_copy(
                tabT.at[:, pl.ds(0, 128)], Sg.at[b], rsem[b]
            ).wait()

        def fire_write(j, b):
            pltpu.async_copy(T.at[b], I.at[pl.ds(j * 64, 64)], wsem[b])

        def wait_write(b):
            pltpu.make_async_copy(
                T.at[b], I.at[pl.ds(0, 64)], wsem[b]
            ).wait()

        def transpose(b, npairs):
            PU = 4

            def pbody(pq, carry):
                for pu in range(PU):
                    p = pq * PU + pu
                    for g in range(8):
                        rows = lax.iota(jnp.int32, 16) + (g % 4) * 16
                        col = jnp.full((16,), 2, jnp.int32) * p + (
                            1 if g >= 4 else 0
                        )
                        vals = plsc.load_gather(Sg.at[b], [rows, col])
                        T[b, p, pl.ds(g * 16, 16)] = vals
                return carry

            lax.fori_loop(0, npairs // PU, pbody, 0)

        fire_read(j0, 0)
        fire_read(j0 + 1, 1)
        for b in range(2):
            wait_read(b)
            transpose(b, 64)
            fire_write(j0 + b, b)
            fire_read(j0 + b + 2, b)

        def body(q, carry):
            for b in range(2):
                t = 2 * q + b
                wait_read(b)
                wait_write(b)
                transpose(b, 64)
                fire_write(j0 + t, b)
                fire_read(j0 + t + 2, b)
            return carry

        lax.fori_loop(1, n_pairs_iters - 1, body, 0)

        for b in range(2):
            t = per_w - 2 + b
            wait_read(b)
            wait_write(b)
            transpose(b, 64)
            fire_write(j0 + t, b)
        for b in range(2):
            wait_write(b)

        @pl.when(wid < extras)
        def _():
            j = NW * per_w + wid
            pltpu.sync_copy(tabT.at[:, pl.ds(j * 128, 128)], Sg.at[0])
            transpose(0, 64)
            pltpu.sync_copy(T.at[0], I.at[pl.ds(j * 64, 64)])

        @pl.when(wid == NW - 1)
        def _():
            pltpu.sync_copy(tailT, Sg.at[0])
            transpose(0, TAIL // 2)
            pltpu.sync_copy(
                T.at[0, pl.ds(0, TAIL // 2)],
                I.at[pl.ds(NBLK * 64, TAIL // 2)],
            )

    return ka


def _gather_call(S, D, B0, NP):
    mesh = plsc.VectorSubcoreMesh(core_axis_name="c", subcore_axis_name="s")

    @functools.partial(
        pl.kernel,
        mesh=mesh,
        out_type=jax.ShapeDtypeStruct((S, D, B0), jnp.float32),
        compiler_params=pltpu.CompilerParams(needs_layout_passes=False),
        scratch_types=[
            pltpu.VMEM((S, 128), jnp.int32),
            pltpu.VMEM((2, 128), jnp.int32),
            pltpu.VMEM((2, 128), jnp.int32),
            pltpu.VMEM((2, 128, 2 * D), jnp.float32),
            pltpu.VMEM((2, D, 128), jnp.float32),
            pltpu.SemaphoreType.DMA,
            pltpu.SemaphoreType.DMA,
            pltpu.SemaphoreType.DMA,
            pltpu.SemaphoreType.DMA,
        ],
    )
    def kb(I, dataT, out, idxP, pidx, par64, G, T, g0, g1, o0, o1):
        gsem = (g0, g1)
        osem = (o0, o1)
        wid = _mesh_wid()
        c0 = pl.multiple_of(wid * 128, 128)
        pltpu.sync_copy(dataT.at[:, pl.ds(c0, 128)], idxP)

        def compute_idx(s, b):
            for g in range(8):
                v = idxP[s, pl.ds(g * 16, 16)]
                pidx[b, pl.ds(g * 16, 16)] = lax.shift_right_logical(v, 1)
                par64[b, pl.ds(g * 16, 16)] = lax.shift_left(v & 1, 6)

        def fire_gather(b):
            pltpu.async_copy(I.at[pidx.at[b]], G.at[b], gsem[b])

        def wait_gather(b):
            pltpu.make_async_copy(I.at[pidx.at[b]], G.at[b], gsem[b]).wait()

        def fire_out(s, b):
            pltpu.async_copy(T.at[b], out.at[s, :, pl.ds(c0, 128)], osem[b])

        def wait_out(b):
            pltpu.make_async_copy(
                T.at[b], out.at[0, :, pl.ds(c0, 128)], osem[b]
            ).wait()

        def select_transpose(b):
            DU = 8
            for g in range(8):
                rows = lax.iota(jnp.int32, 16) + g * 16
                pv = par64[b, pl.ds(g * 16, 16)]

                def dbody(dq, carry, rows=rows, pv=pv):
                    for du in range(DU):
                        d = dq * DU + du
                        vals = plsc.load_gather(G.at[b], [rows, pv + d])
                        T[b, d, pl.ds(g * 16, 16)] = vals
                    return carry

                lax.fori_loop(0, D // DU, dbody, 0)

        for b in range(2):
            compute_idx(b, b)
            fire_gather(b)
        for b in range(2):
            wait_gather(b)
            select_transpose(b)
            fire_out(b, b)
            compute_idx(b + 2, b)
            fire_gather(b)

        def body(q, carry):
            for b in range(2):
                t = 2 * q + b
                wait_gather(b)
                wait_out(b)
                select_transpose(b)
                fire_out(t, b)
                compute_idx(t + 2, b)
                fire_gather(b)
            return carry

        lax.fori_loop(1, S // 2 - 1, body, 0)

        for b in range(2):
            t = S - 2 + b
            wait_gather(b)
            wait_out(b)
            select_transpose(b)
            fire_out(t, b)
        for b in range(2):
            wait_out(b)

    return kb


def kernel(data, table):
    B0, S = data.shape
    V, D = table.shape
    NP = (V - 1) // 2
    NBLK = (V - 1) // 128
    tailT = jnp.pad(table.T[:, NBLK * 128:], ((0, 0), (0, 128 - (V - NBLK * 128))))
    I = _transpose_call(D, V)(table.T, tailT)
    out = _gather_call(S, D, B0, NP)(I, data.T)
    return jnp.transpose(out, (2, 0, 1))


# speedup vs baseline: 1.8643x; 1.8643x over previous
import functools

import jax
import jax.numpy as jnp
from jax import lax
from jax.experimental import pallas as pl
from jax.experimental.pallas import tpu as pltpu, tpu_sc as plsc

NW = 32


def _mesh_wid():
    info = plsc.get_sparse_core_info()
    wid = lax.axis_index("s") * info.num_cores + lax.axis_index("c")
    return wid


def _transpose_call(D, V):
    NP = (V - 1) // 2
    NBLK = (V - 1) // 128
    TAIL = (V - 1) - NBLK * 128
    per_w = NBLK // NW
    extras = NBLK - per_w * NW
    n_pairs_iters = per_w // 2
    mesh = plsc.VectorSubcoreMesh(core_axis_name="c", subcore_axis_name="s")

    @functools.partial(
        pl.kernel,
        mesh=mesh,
        out_type=jax.ShapeDtypeStruct((NP, 2 * D), jnp.float32),
        compiler_params=pltpu.CompilerParams(needs_layout_passes=False),
        scratch_types=[
            pltpu.VMEM((2, D, 128), jnp.float32),
            pltpu.VMEM((2, 64, 128), jnp.float32),
            pltpu.SemaphoreType.DMA,
            pltpu.SemaphoreType.DMA,
            pltpu.SemaphoreType.DMA,
            pltpu.SemaphoreType.DMA,
        ],
    )
    def ka(tabT, tailT, I, Sg, T, r0, r1, w0, w1):
        rsem = (r0, r1)
        wsem = (w0, w1)
        wid = _mesh_wid()
        j0 = wid * per_w

        def fire_read(j, b):
            pltpu.async_copy(tabT.at[:, pl.ds(j * 128, 128)], Sg.at[b], rsem[b])

        def wait_read(b):
            pltpu.make_async_copy(
                tabT.at[:, pl.ds(0, 128)], Sg.at[b], rsem[b]
            ).wait()

        def fire_write(j, b):
            pltpu.async_copy(T.at[b], I.at[pl.ds(j * 64, 64)], wsem[b])

        def wait_write(b):
            pltpu.make_async_copy(
                T.at[b], I.at[pl.ds(0, 64)], wsem[b]
            ).wait()

        def transpose(b, npairs):
            @plsc.parallel_loop(0, npairs, 1, unroll=4)
            def _(p):
                for g in range(8):
                    rows = lax.iota(jnp.int32, 16) + (g % 4) * 16
                    col = jnp.full((16,), 2, jnp.int32) * p + (
                        1 if g >= 4 else 0
                    )
                    vals = plsc.load_gather(Sg.at[b], [rows, col])
                    T[b, p, pl.ds(g * 16, 16)] = vals

        fire_read(j0, 0)
        fire_read(j0 + 1, 1)
        for b in range(2):
            wait_read(b)
            transpose(b, 64)
            fire_write(j0 + b, b)
            fire_read(j0 + b + 2, b)

        def body(q, carry):
            for b in range(2):
                t = 2 * q + b
                wait_read(b)
                wait_write(b)
                transpose(b, 64)
                fire_write(j0 + t, b)
                fire_read(j0 + t + 2, b)
            return carry

        lax.fori_loop(1, n_pairs_iters - 1, body, 0)

        for b in range(2):
            t = per_w - 2 + b
            wait_read(b)
            wait_write(b)
            transpose(b, 64)
            fire_write(j0 + t, b)
        for b in range(2):
            wait_write(b)

        @pl.when(wid < extras)
        def _():
            j = NW * per_w + wid
            pltpu.sync_copy(tabT.at[:, pl.ds(j * 128, 128)], Sg.at[0])
            transpose(0, 64)
            pltpu.sync_copy(T.at[0], I.at[pl.ds(j * 64, 64)])

        @pl.when(wid == NW - 1)
        def _():
            pltpu.sync_copy(tailT, Sg.at[0])
            transpose(0, TAIL // 2)
            pltpu.sync_copy(
                T.at[0, pl.ds(0, TAIL // 2)],
                I.at[pl.ds(NBLK * 64, TAIL // 2)],
            )

    return ka


def _gather_call(S, D, B0, NP):
    mesh = plsc.VectorSubcoreMesh(core_axis_name="c", subcore_axis_name="s")

    @functools.partial(
        pl.kernel,
        mesh=mesh,
        out_type=jax.ShapeDtypeStruct((S, D, B0), jnp.float32),
        compiler_params=pltpu.CompilerParams(needs_layout_passes=False),
        scratch_types=[
            pltpu.VMEM((S, 128), jnp.int32),
            pltpu.VMEM((2, 128), jnp.int32),
            pltpu.VMEM((2, 128), jnp.int32),
            pltpu.VMEM((2, 128, 2 * D), jnp.float32),
            pltpu.VMEM((2, D, 128), jnp.float32),
            pltpu.SemaphoreType.DMA,
            pltpu.SemaphoreType.DMA,
            pltpu.SemaphoreType.DMA,
            pltpu.SemaphoreType.DMA,
        ],
    )
    def kb(I, dataT, out, idxP, pidx, par64, G, T, g0, g1, o0, o1):
        gsem = (g0, g1)
        osem = (o0, o1)
        wid = _mesh_wid()
        c0 = pl.multiple_of(wid * 128, 128)
        pltpu.sync_copy(dataT.at[:, pl.ds(c0, 128)], idxP)

        def compute_idx(s, b):
            for g in range(8):
                v = idxP[s, pl.ds(g * 16, 16)]
                pidx[b, pl.ds(g * 16, 16)] = lax.shift_right_logical(v, 1)
                par64[b, pl.ds(g * 16, 16)] = lax.shift_left(v & 1, 6)

        def fire_gather(b):
            pltpu.async_copy(I.at[pidx.at[b]], G.at[b], gsem[b])

        def wait_gather(b):
            pltpu.make_async_copy(I.at[pidx.at[b]], G.at[b], gsem[b]).wait()

        def fire_out(s, b):
            pltpu.async_copy(T.at[b], out.at[s, :, pl.ds(c0, 128)], osem[b])

        def wait_out(b):
            pltpu.make_async_copy(
                T.at[b], out.at[0, :, pl.ds(c0, 128)], osem[b]
            ).wait()

        def select_transpose(b):
            for g in range(8):
                rows = lax.iota(jnp.int32, 16) + g * 16
                pv = par64[b, pl.ds(g * 16, 16)]

                @plsc.parallel_loop(0, D, 1, unroll=8)
                def _(d, rows=rows, pv=pv):
                    vals = plsc.load_gather(G.at[b], [rows, pv + d])
                    T[b, d, pl.ds(g * 16, 16)] = vals

        for b in range(2):
            compute_idx(b, b)
            fire_gather(b)
        for b in range(2):
            wait_gather(b)
            select_transpose(b)
            fire_out(b, b)
            compute_idx(b + 2, b)
            fire_gather(b)

        def body(q, carry):
            for b in range(2):
                t = 2 * q + b
                wait_gather(b)
                wait_out(b)
                select_transpose(b)
                fire_out(t, b)
                compute_idx(t + 2, b)
                fire_gather(b)
            return carry

        lax.fori_loop(1, S // 2 - 1, body, 0)

        for b in range(2):
            t = S - 2 + b
            wait_gather(b)
            wait_out(b)
            select_transpose(b)
            fire_out(t, b)
        for b in range(2):
            wait_out(b)

    return kb


def kernel(data, table):
    B0, S = data.shape
    V, D = table.shape
    NP = (V - 1) // 2
    NBLK = (V - 1) // 128
    tailT = jnp.pad(table.T[:, NBLK * 128:], ((0, 0), (0, 128 - (V - NBLK * 128))))
    I = _transpose_call(D, V)(table.T, tailT)
    out = _gather_call(S, D, B0, NP)(I, data.T)
    return jnp.transpose(out, (2, 0, 1))


# speedup vs baseline: 2.4437x; 1.3107x over previous
import functools

import jax
import jax.numpy as jnp
from jax import lax
from jax.experimental import pallas as pl
from jax.experimental.pallas import tpu as pltpu, tpu_sc as plsc

NW = 32


def _mesh_wid():
    info = plsc.get_sparse_core_info()
    wid = lax.axis_index("s") * info.num_cores + lax.axis_index("c")
    return wid


def _transpose_call(D, V):
    NP = (V - 1) // 2
    NBLK = (V - 1) // 128
    TAIL = (V - 1) - NBLK * 128
    per_w = NBLK // NW
    extras = NBLK - per_w * NW
    n_pairs_iters = per_w // 2
    mesh = plsc.VectorSubcoreMesh(core_axis_name="c", subcore_axis_name="s")

    @functools.partial(
        pl.kernel,
        mesh=mesh,
        out_type=jax.ShapeDtypeStruct((NP, 2 * D), jnp.float32),
        compiler_params=pltpu.CompilerParams(needs_layout_passes=False),
        scratch_types=[
            pltpu.VMEM((2, D, 128), jnp.float32),
            pltpu.VMEM((2, 64, 128), jnp.float32),
            pltpu.SemaphoreType.DMA,
            pltpu.SemaphoreType.DMA,
            pltpu.SemaphoreType.DMA,
            pltpu.SemaphoreType.DMA,
        ],
    )
    def ka(tabT, tailT, I, Sg, T, r0, r1, w0, w1):
        rsem = (r0, r1)
        wsem = (w0, w1)
        wid = _mesh_wid()
        j0 = wid * per_w

        def fire_read(j, b):
            pltpu.async_copy(tabT.at[:, pl.ds(j * 128, 128)], Sg.at[b], rsem[b])

        def wait_read(b):
            pltpu.make_async_copy(
                tabT.at[:, pl.ds(0, 128)], Sg.at[b], rsem[b]
            ).wait()

        def fire_write(j, b):
            pltpu.async_copy(T.at[b], I.at[pl.ds(j * 64, 64)], wsem[b])

        def wait_write(b):
            pltpu.make_async_copy(
                T.at[b], I.at[pl.ds(0, 64)], wsem[b]
            ).wait()

        def transpose(b, npairs):
            @plsc.parallel_loop(0, npairs, 1, unroll=4)
            def _(p):
                for g in range(8):
                    rows = lax.iota(jnp.int32, 16) + (g % 4) * 16
                    col = jnp.full((16,), 2, jnp.int32) * p + (
                        1 if g >= 4 else 0
                    )
                    vals = plsc.load_gather(Sg.at[b], [rows, col])
                    T[b, p, pl.ds(g * 16, 16)] = vals

        fire_read(j0, 0)
        fire_read(j0 + 1, 1)
        for b in range(2):
            wait_read(b)
            transpose(b, 64)
            fire_write(j0 + b, b)
            fire_read(j0 + b + 2, b)

        def body(q, carry):
            for b in range(2):
                t = 2 * q + b
                wait_read(b)
                wait_write(b)
                transpose(b, 64)
                fire_write(j0 + t, b)
                fire_read(j0 + t + 2, b)
            return carry

        lax.fori_loop(1, n_pairs_iters - 1, body, 0)

        for b in range(2):
            t = per_w - 2 + b
            wait_read(b)
            wait_write(b)
            transpose(b, 64)
            fire_write(j0 + t, b)
        for b in range(2):
            wait_write(b)

        @pl.when(wid < extras)
        def _():
            j = NW * per_w + wid
            pltpu.sync_copy(tabT.at[:, pl.ds(j * 128, 128)], Sg.at[0])
            transpose(0, 64)
            pltpu.sync_copy(T.at[0], I.at[pl.ds(j * 64, 64)])

        @pl.when(wid == NW - 1)
        def _():
            pltpu.sync_copy(tailT, Sg.at[0])
            transpose(0, TAIL // 2)
            pltpu.sync_copy(
                T.at[0, pl.ds(0, TAIL // 2)],
                I.at[pl.ds(NBLK * 64, TAIL // 2)],
            )

    return ka


def _gather_call(B, D, NP):
    IDXW = 128
    rows_per_w = B // (NW * IDXW)
    K = 5
    n_chunks = rows_per_w // K
    C = K * IDXW
    mesh = plsc.VectorSubcoreMesh(core_axis_name="c", subcore_axis_name="s")

    @functools.partial(
        pl.kernel,
        mesh=mesh,
        out_type=jax.ShapeDtypeStruct((B, 2 * D), jnp.float32),
        compiler_params=pltpu.CompilerParams(use_tc_tiling_on_sc=False),
        scratch_types=[
            pltpu.VMEM((rows_per_w, IDXW), jnp.int32),
            pltpu.VMEM((2, C, D), jnp.float32),
            pltpu.SemaphoreType.DMA,
            pltpu.SemaphoreType.DMA,
            pltpu.SemaphoreType.DMA,
            pltpu.SemaphoreType.DMA,
        ],
    )
    def kb(IR, idx_hbm, out, idx_v, G, g0, g1, o0, o1):
        gsem = (g0, g1)
        osem = (o0, o1)
        wid = _mesh_wid()
        row0 = wid * rows_per_w

        pltpu.sync_copy(idx_hbm.at[pl.ds(row0, rows_per_w)], idx_v)

        def fire(t, b):
            for j in range(K):
                pltpu.async_copy(
                    IR.at[idx_v.at[t * K + j]],
                    G.at[b, pl.ds(j * IDXW, IDXW)],
                    gsem[b],
                )

        def drain_gather(b):
            for _ in range(K):
                pltpu.make_async_copy(
                    IR.at[idx_v.at[0]], G.at[b, pl.ds(0, IDXW)], gsem[b]
                ).wait()

        def start_out(t, b):
            pltpu.async_copy(
                G.at[b],
                out.at[pl.ds((row0 + t * K) * IDXW, C), pl.ds(0, D)],
                osem[b],
            )

        def drain_out(b):
            pltpu.make_async_copy(
                G.at[b], out.at[pl.ds(0, C), pl.ds(0, D)], osem[b]
            ).wait()

        fire(0, 0)
        fire(1, 1)

        def body(q, carry):
            t0 = 2 * q
            for b in range(2):
                drain_gather(b)
                start_out(t0 + b, b)
            for b in range(2):
                drain_out(b)
                fire(t0 + 2 + b, b)
            return carry

        lax.fori_loop(0, n_chunks // 2 - 1, body, 0)

        t0 = n_chunks - 2
        for b in range(2):
            drain_gather(b)
            start_out(t0 + b, b)
        for b in range(2):
            drain_out(b)

    return kb


def kernel(data, table):
    B0, S = data.shape
    V, D = table.shape
    B = B0 * S
    NP = (V - 1) // 2
    NBLK = (V - 1) // 128
    tailT = jnp.pad(table.T[:, NBLK * 128:], ((0, 0), (0, 128 - (V - NBLK * 128))))
    I = _transpose_call(D, V)(table.T, tailT)
    IR = I.reshape(2 * NP, D)
    outP = _gather_call(B, D, NP)(IR, data.reshape(B // 128, 128))
    return outP[:, :D].reshape(B0, S, D)
